# baseline (speedup 1.0000x reference)
import functools

import jax
import jax.numpy as jnp
from jax import lax
from jax.experimental import pallas as pl
from jax.experimental.pallas import tpu as pltpu
from jax.experimental.pallas import tpu_sc as plsc

N = 10000
D = 128
E = 320000
BN_EPS_CONST = 1e-5

NC = 2
NS = 16
NW = NC * NS
CK = 80
CHUNKS_PER_W = E // NW // CK
GE = 2000
G = 25
NG = CHUNKS_PER_W // G

ROW_STRIDE = 624
ZCHUNK = 80
NZ = 8


def _sc_scatter_body(x_hbm, src_hbm, dst_hbm, out_hbm, acc, src_v, dst_v,
                     src_w, dst_w,
                     rows_0, rows_1, rows_2, gsem0, gsem1, gsem2,
                     ssem0, ssem1, ssem2, isem):
    cid = lax.axis_index("c")
    sid = lax.axis_index("s")
    wid = sid * NC + cid
    row0 = sid * ROW_STRIDE

    bufs = (rows_0, rows_1, rows_2)
    gsems = (gsem0, gsem1, gsem2)
    ssems = (ssem0, ssem1, ssem2)
    idx_ab = ((src_v, dst_v), (src_w, dst_w))

    H = CK // 2

    def start_gather(sv, j, b):
        pltpu.async_copy(x_hbm.at[sv.at[pl.ds(j * CK, H)]],
                         bufs[b].at[pl.ds(0, H)], gsems[b])
        pltpu.async_copy(x_hbm.at[sv.at[pl.ds(j * CK + H, H)]],
                         bufs[b].at[pl.ds(H, H)], gsems[b])

    def wait_gather(b):
        pltpu.make_async_copy(x_hbm.at[src_v.at[pl.ds(0, H)]],
                              bufs[b].at[pl.ds(0, H)], gsems[b]).wait()
        pltpu.make_async_copy(x_hbm.at[src_v.at[pl.ds(0, H)]],
                              bufs[b].at[pl.ds(H, H)], gsems[b]).wait()

    def start_scatter(dv, j, b):
        pltpu.async_copy(bufs[b], acc.at[dv.at[j]], ssems[b], add=True)

    def wait_scatter(b):
        pltpu.make_async_copy(bufs[b], acc.at[dst_v.at[0]], ssems[b]).wait()

    def load_idx(g, sv, dv):
        base = wid * (CHUNKS_PER_W * CK) + g * GE
        pltpu.async_copy(src_hbm.at[pl.ds(base, GE)], sv, isem)
        for k in range(G):
            pltpu.async_copy(dst_hbm.at[pl.ds(base + k * CK, CK)],
                             dv.at[k], isem)

    def wait_idx(sv, dv):
        pltpu.make_async_copy(src_hbm.at[pl.ds(0, GE)], sv, isem).wait()
        pltpu.make_async_copy(src_hbm.at[pl.ds(0, GE)], sv, isem).wait()

    def edge_loop(sv, dv):
        wait_gather(0)
        start_scatter(dv, 0, 0)

        def edge_body(t, _):
            for i in range(3):
                j = 3 * t + 1 + i
                s = (1 + i) % 3
                wait_scatter((s + 2) % 3)

                @pl.when(j + 2 <= G - 1)
                def _():
                    start_gather(sv, j + 2, (s + 2) % 3)
                wait_gather(s)
                start_scatter(dv, j, s)
            return 0
        lax.fori_loop(0, (G - 1) // 3, edge_body, 0, unroll=False)

    sv, dv = idx_ab[0]
    load_idx(0, sv, dv)

    def zbody(i, _):
        r = i // (D // 16)
        c = (i % (D // 16)) * 16
        rows_0[r, pl.ds(c, 16)] = jnp.zeros((16,), jnp.float32)
        return 0
    lax.fori_loop(0, ZCHUNK * (D // 16), zbody, 0)

    def zcopy(i, _):
        pltpu.async_copy(rows_0, acc.at[pl.ds(row0 + i * ZCHUNK, ZCHUNK)],
                         ssem0)
        return 0
    lax.fori_loop(0, NZ, zcopy, 0)
    wait_idx(sv, dv)
    load_idx(1, *idx_ab[1])
    start_gather(sv, 1, 1)
    start_gather(sv, 2, 2)

    def zwait(i, _):
        pltpu.make_async_copy(rows_0, acc.at[pl.ds(row0, ZCHUNK)],
                              ssem0).wait()
        return 0
    lax.fori_loop(0, NZ, zwait, 0)
    plsc.subcore_barrier()
    start_gather(sv, 0, 0)
    edge_loop(sv, dv)

    for g in range(1, NG):
        sv, dv = idx_ab[g % 2]
        wait_idx(sv, dv)
        start_gather(sv, 1, 1)
        start_gather(sv, 2, 2)
        wait_scatter(0)
        if g + 1 < NG:
            load_idx(g + 1, *idx_ab[(g + 1) % 2])
        start_gather(sv, 0, 0)
        edge_loop(sv, dv)

    wait_scatter(0)
    plsc.subcore_barrier()

    def ocopy(i, _):
        pltpu.async_copy(acc.at[pl.ds(row0 + i * ZCHUNK, ZCHUNK)],
                         out_hbm.at[cid, pl.ds(row0 + i * ZCHUNK, ZCHUNK)],
                         ssem0)
        return 0
    lax.fori_loop(0, NZ, ocopy, 0)
    def owait(i, _):
        pltpu.make_async_copy(acc.at[pl.ds(row0, ZCHUNK)],
                              out_hbm.at[cid, pl.ds(row0, ZCHUNK)],
                              ssem0).wait()
        return 0
    lax.fori_loop(0, NZ, owait, 0)


_sc_scatter = functools.partial(
    pl.kernel,
    out_type=jax.ShapeDtypeStruct((NC, N, D), jnp.float32),
    mesh=plsc.VectorSubcoreMesh(core_axis_name="c", subcore_axis_name="s"),
    scratch_types=[
        pltpu.VMEM_SHARED((N, D), jnp.float32),
        pltpu.VMEM((GE,), jnp.int32),
        pltpu.VMEM((G, CK), jnp.int32),
        pltpu.VMEM((GE,), jnp.int32),
        pltpu.VMEM((G, CK), jnp.int32),
        pltpu.VMEM((CK, D), jnp.float32),
        pltpu.VMEM((CK, D), jnp.float32),
        pltpu.VMEM((CK, D), jnp.float32),
        pltpu.SemaphoreType.DMA,
        pltpu.SemaphoreType.DMA,
        pltpu.SemaphoreType.DMA,
        pltpu.SemaphoreType.DMA,
        pltpu.SemaphoreType.DMA,
        pltpu.SemaphoreType.DMA,
        pltpu.SemaphoreType.DMA,
    ],
)(_sc_scatter_body)


def _tc_body(p_ref, x_ref, epsb_ref, W1_ref, b1_ref, W2_ref, b2_ref,
             gamma_ref, beta_ref, o_ref):
    agg = p_ref[0] + p_ref[1] + epsb_ref[0, 0] * x_ref[...]
    h1 = jax.lax.dot_general(agg, W1_ref[...], (((1,), (1,)), ((), ())),
                             preferred_element_type=jnp.float32)
    h1 = jnp.maximum(h1 + b1_ref[...], 0.0)
    h2 = jax.lax.dot_general(h1, W2_ref[...], (((1,), (1,)), ((), ())),
                             preferred_element_type=jnp.float32)
    h2 = h2 + b2_ref[...]
    mean = jnp.mean(h2, axis=0, keepdims=True)
    var = jnp.mean(h2 * h2, axis=0, keepdims=True) - mean * mean
    inv = jax.lax.rsqrt(var + BN_EPS_CONST)
    o_ref[...] = jnp.maximum(
        (h2 - mean) * (inv * gamma_ref[...]) + beta_ref[...], 0.0)


def kernel(x, edge_index, W1, b1, W2, b2, gamma, beta, epsilon):
    partials = _sc_scatter(x, edge_index[0], edge_index[1])

    epsb = jnp.reshape(1.0 + epsilon, (1, 1)).astype(jnp.float32)
    out = pl.pallas_call(
        _tc_body,
        out_shape=jax.ShapeDtypeStruct((N, D), jnp.float32),
    )(partials, x, epsb, W1, b1.reshape(1, D), W2, b2.reshape(1, D),
      gamma.reshape(1, D), beta.reshape(1, D))
    return out


# speedup vs baseline: 1.0652x; 1.0652x over previous
import functools

import jax
import jax.numpy as jnp
from jax import lax
from jax.experimental import pallas as pl
from jax.experimental.pallas import tpu as pltpu
from jax.experimental.pallas import tpu_sc as plsc

N = 10000
D = 128
E = 320000
BN_EPS_CONST = 1e-5

NC = 2
NS = 16
NW = NC * NS
CK = 80
CHUNKS_PER_W = E // NW // CK
G = 25
NG = CHUNKS_PER_W // G

ROW_STRIDE = 624
ZCHUNK = 80
NZ = 8


def _sc_scatter_body(x_hbm, ei_hbm, out_hbm, acc, src_v, dst_v,
                     src_w, dst_w,
                     rows_0, rows_1, rows_2, gsem0, gsem1, gsem2,
                     ssem0, ssem1, ssem2, isem):
    cid = lax.axis_index("c")
    sid = lax.axis_index("s")
    wid = sid * NC + cid
    row0 = sid * ROW_STRIDE

    bufs = (rows_0, rows_1, rows_2)
    gsems = (gsem0, gsem1, gsem2)
    ssems = (ssem0, ssem1, ssem2)
    idx_ab = ((src_v, dst_v), (src_w, dst_w))

    H = CK // 2

    def start_gather(sv, j, b):
        pltpu.async_copy(x_hbm.at[sv.at[j, pl.ds(0, H)]],
                         bufs[b].at[pl.ds(0, H)], gsems[b])
        pltpu.async_copy(x_hbm.at[sv.at[j, pl.ds(H, H)]],
                         bufs[b].at[pl.ds(H, H)], gsems[b])

    def wait_gather(b):
        pltpu.make_async_copy(x_hbm.at[src_v.at[0, pl.ds(0, H)]],
                              bufs[b].at[pl.ds(0, H)], gsems[b]).wait()
        pltpu.make_async_copy(x_hbm.at[src_v.at[0, pl.ds(H, H)]],
                              bufs[b].at[pl.ds(H, H)], gsems[b]).wait()

    def start_scatter(dv, j, b):
        pltpu.async_copy(bufs[b], acc.at[dv.at[j]], ssems[b], add=True)

    def wait_scatter(b):
        pltpu.make_async_copy(bufs[b], acc.at[dst_v.at[0]], ssems[b]).wait()

    def load_idx(g, sv, dv):
        pltpu.async_copy(ei_hbm.at[0, wid, g], sv, isem)
        pltpu.async_copy(ei_hbm.at[1, wid, g], dv, isem)

    def wait_idx(sv, dv):
        pltpu.make_async_copy(ei_hbm.at[0, wid, 0], sv, isem).wait()
        pltpu.make_async_copy(ei_hbm.at[1, wid, 0], dv, isem).wait()

    def edge_loop(sv, dv):
        wait_gather(0)
        start_scatter(dv, 0, 0)

        def edge_body(t, _):
            for i in range(3):
                j = 3 * t + 1 + i
                s = (1 + i) % 3
                wait_scatter((s + 2) % 3)

                @pl.when(j + 2 <= G - 1)
                def _():
                    start_gather(sv, j + 2, (s + 2) % 3)
                wait_gather(s)
                start_scatter(dv, j, s)
            return 0
        lax.fori_loop(0, (G - 1) // 3, edge_body, 0, unroll=False)

    sv, dv = idx_ab[0]
    load_idx(0, sv, dv)

    def zbody(i, _):
        r = i // (D // 16)
        c = (i % (D // 16)) * 16
        rows_0[r, pl.ds(c, 16)] = jnp.zeros((16,), jnp.float32)
        return 0
    lax.fori_loop(0, ZCHUNK * (D // 16), zbody, 0)

    def zcopy(i, _):
        pltpu.async_copy(rows_0, acc.at[pl.ds(row0 + i * ZCHUNK, ZCHUNK)],
                         ssem0)
        return 0
    lax.fori_loop(0, NZ, zcopy, 0)
    wait_idx(sv, dv)
    load_idx(1, *idx_ab[1])
    start_gather(sv, 1, 1)
    start_gather(sv, 2, 2)

    def zwait(i, _):
        pltpu.make_async_copy(rows_0, acc.at[pl.ds(row0, ZCHUNK)],
                              ssem0).wait()
        return 0
    lax.fori_loop(0, NZ, zwait, 0)
    plsc.subcore_barrier()
    start_gather(sv, 0, 0)
    edge_loop(sv, dv)

    for g in range(1, NG):
        sv, dv = idx_ab[g % 2]
        wait_idx(sv, dv)
        start_gather(sv, 1, 1)
        start_gather(sv, 2, 2)
        wait_scatter(0)
        if g + 1 < NG:
            load_idx(g + 1, *idx_ab[(g + 1) % 2])
        start_gather(sv, 0, 0)
        edge_loop(sv, dv)

    wait_scatter(0)
    plsc.subcore_barrier()

    def ocopy(i, _):
        pltpu.async_copy(acc.at[pl.ds(row0 + i * ZCHUNK, ZCHUNK)],
                         out_hbm.at[cid, pl.ds(row0 + i * ZCHUNK, ZCHUNK)],
                         ssem0)
        return 0
    lax.fori_loop(0, NZ, ocopy, 0)
    def owait(i, _):
        pltpu.make_async_copy(acc.at[pl.ds(row0, ZCHUNK)],
                              out_hbm.at[cid, pl.ds(row0, ZCHUNK)],
                              ssem0).wait()
        return 0
    lax.fori_loop(0, NZ, owait, 0)


_sc_scatter = functools.partial(
    pl.kernel,
    out_type=jax.ShapeDtypeStruct((NC, N, D), jnp.float32),
    mesh=plsc.VectorSubcoreMesh(core_axis_name="c", subcore_axis_name="s"),
    scratch_types=[
        pltpu.VMEM_SHARED((N, D), jnp.float32),
        pltpu.VMEM((G, CK), jnp.int32),
        pltpu.VMEM((G, CK), jnp.int32),
        pltpu.VMEM((G, CK), jnp.int32),
        pltpu.VMEM((G, CK), jnp.int32),
        pltpu.VMEM((CK, D), jnp.float32),
        pltpu.VMEM((CK, D), jnp.float32),
        pltpu.VMEM((CK, D), jnp.float32),
        pltpu.SemaphoreType.DMA,
        pltpu.SemaphoreType.DMA,
        pltpu.SemaphoreType.DMA,
        pltpu.SemaphoreType.DMA,
        pltpu.SemaphoreType.DMA,
        pltpu.SemaphoreType.DMA,
        pltpu.SemaphoreType.DMA,
    ],
)(_sc_scatter_body)


def _tc_body(p_ref, x_ref, epsb_ref, W1_ref, b1_ref, W2_ref, b2_ref,
             gamma_ref, beta_ref, o_ref):
    agg = p_ref[0] + p_ref[1] + (1.0 + epsb_ref[0, 0]) * x_ref[...]
    h1 = jax.lax.dot_general(agg, W1_ref[...], (((1,), (1,)), ((), ())),
                             preferred_element_type=jnp.float32)
    h1 = jnp.maximum(h1 + b1_ref[...], 0.0)
    h2 = jax.lax.dot_general(h1, W2_ref[...], (((1,), (1,)), ((), ())),
                             preferred_element_type=jnp.float32)
    h2 = h2 + b2_ref[...]
    mean = jnp.mean(h2, axis=0, keepdims=True)
    var = jnp.mean(h2 * h2, axis=0, keepdims=True) - mean * mean
    inv = jax.lax.rsqrt(var + BN_EPS_CONST)
    o_ref[...] = jnp.maximum(
        (h2 - mean) * (inv * gamma_ref[...]) + beta_ref[...], 0.0)


def kernel(x, edge_index, W1, b1, W2, b2, gamma, beta, epsilon):
    ei = edge_index.reshape(2, NW, NG, G, CK)
    partials = _sc_scatter(x, ei)

    epsb = epsilon.reshape(1, 1)
    out = pl.pallas_call(
        _tc_body,
        out_shape=jax.ShapeDtypeStruct((N, D), jnp.float32),
    )(partials, x, epsb, W1, b1.reshape(1, D), W2, b2.reshape(1, D),
      gamma.reshape(1, D), beta.reshape(1, D))
    return out
